# baseline (speedup 1.0000x reference)
import functools

import jax
import jax.numpy as jnp
from jax import lax
from jax.experimental import pallas as pl
from jax.experimental.pallas import tpu as pltpu
from jax.experimental.pallas import tpu_sc as plsc

N = 4096
D = 128
NC, NS = 2, 16
NW = NC * NS
BLK = 512
CH = 128
NCHUNK = N // CH
NBUF = 4
NITER = NCHUNK // NBUF


def _rank_body(xrow_ref, idx_ref):
    xrow = xrow_ref[...]
    xcol = jnp.transpose(xrow)
    jrow = lax.broadcasted_iota(jnp.int32, (1, N), 1)

    ones = jnp.ones((1, BLK), jnp.bfloat16)
    gi_l = lax.broadcasted_iota(jnp.int32, (BLK, BLK), 0)
    gj_l = lax.broadcasted_iota(jnp.int32, (BLK, BLK), 1)
    tri = gi_l < gj_l
    NB = N // BLK
    parts = []
    for bj in range(NB):
        xr = xrow[:, bj * BLK:(bj + 1) * BLK]
        accf = jnp.zeros((1, BLK), jnp.float32)
        for bi in range(NB):
            a = xcol[bi * BLK:(bi + 1) * BLK, :]
            if bi < bj:
                mb = a <= xr
            elif bi > bj:
                mb = a < xr
            else:
                mb = (a < xr) | ((a == xr) & tri)
            accf = accf + jax.lax.dot_general(
                ones, mb.astype(jnp.bfloat16), (((1,), (0,)), ((), ())),
                preferred_element_type=jnp.float32)
        parts.append(accf)
    acc = jnp.concatenate(parts, axis=1).astype(jnp.int32)

    r = jnp.reshape(acc, (NCHUNK, CH))
    woff = lax.broadcasted_iota(jnp.int32, (NW, 1, 1), 0) * N
    idx_ref[...] = r[None, :, :] + woff


def _rank_indices(x_flat):
    return pl.pallas_call(
        _rank_body,
        out_shape=jax.ShapeDtypeStruct((NW, NCHUNK, CH), jnp.int32),
    )(x_flat.reshape(1, N))


RCH = 256
NRC = N // RCH


def _scatter_body(yflat, idxmat, out, idx_v, buf0, buf1,
                  gsem0, gsem1, osem0, osem1):
    w = lax.axis_index("s") * NC + lax.axis_index("c")
    pltpu.sync_copy(idxmat.at[w], idx_v)

    bufs = (buf0, buf1)
    gsems = (gsem0, gsem1)
    osems = (osem0, osem1)

    def fire_g(r, j):
        pltpu.async_copy(yflat.at[pl.ds(w * N + r * RCH, RCH)], bufs[j],
                         gsems[j])

    def wait_g(r, j):
        pltpu.make_async_copy(yflat.at[pl.ds(w * N + r * RCH, RCH)], bufs[j],
                              gsems[j]).wait()

    def fire_o(r, j, h):
        pltpu.async_copy(bufs[j].at[pl.ds(h * CH, CH)],
                         out.at[idx_v.at[2 * r + h]], osems[j])

    def wait_o(r, j, h):
        pltpu.make_async_copy(bufs[j].at[pl.ds(h * CH, CH)],
                              out.at[idx_v.at[2 * r + h]], osems[j]).wait()

    NB = 2
    NIT = NRC // NB
    for j in range(NB):
        fire_g(j, j)

    def body(i, _):
        r = i * NB
        for j in range(NB):
            wait_g(r + j, j)
            fire_o(r + j, j, 0)
            fire_o(r + j, j, 1)
        for j in range(NB):
            wait_o(r + j, j, 0)
            wait_o(r + j, j, 1)

            @pl.when(i < NIT - 1)
            def _():
                fire_g(r + NB + j, j)
        return 0

    lax.fori_loop(0, NIT, body, 0)


def _scatter_rows(yflat, idxmat):
    mesh = plsc.VectorSubcoreMesh(core_axis_name="c", subcore_axis_name="s")
    return pl.kernel(
        _scatter_body,
        out_type=jax.ShapeDtypeStruct((NW * N, D), jnp.float32),
        mesh=mesh,
        scratch_types=(
            [pltpu.VMEM((NCHUNK, CH), jnp.int32)]
            + [pltpu.VMEM((RCH, D), jnp.float32)] * 2
            + [pltpu.SemaphoreType.DMA] * 4
        ),
    )(yflat, idxmat)


def kernel(x, y):
    idxmat = _rank_indices(x.reshape(N))
    yflat = y.reshape(NW * N, D)
    out = _scatter_rows(yflat, idxmat.reshape(NW, NCHUNK, CH))
    return out.reshape(y.shape)


# speedup vs baseline: 1.0219x; 1.0219x over previous
import functools

import jax
import jax.numpy as jnp
from jax import lax
from jax.experimental import pallas as pl
from jax.experimental.pallas import tpu as pltpu
from jax.experimental.pallas import tpu_sc as plsc

N = 4096
D = 128
NC, NS = 2, 16
NW = NC * NS
BLK = 512
CH = 128
NCHUNK = N // CH
NBUF = 4
NITER = NCHUNK // NBUF


def _rank_body(xrow_ref, idx_ref):
    xrow = xrow_ref[...]
    xcol = jnp.transpose(xrow)
    jrow = lax.broadcasted_iota(jnp.int32, (1, N), 1)

    ones = jnp.ones((1, BLK), jnp.bfloat16)
    gi_l = lax.broadcasted_iota(jnp.int32, (BLK, BLK), 0)
    gj_l = lax.broadcasted_iota(jnp.int32, (BLK, BLK), 1)
    tri = gi_l < gj_l
    NB = N // BLK
    parts = []
    for bj in range(NB):
        xr = xrow[:, bj * BLK:(bj + 1) * BLK]
        accf = jnp.zeros((1, BLK), jnp.float32)
        for bi in range(NB):
            a = xcol[bi * BLK:(bi + 1) * BLK, :]
            if bi < bj:
                mb = a <= xr
            elif bi > bj:
                mb = a < xr
            else:
                mb = (a < xr) | ((a == xr) & tri)
            accf = accf + jax.lax.dot_general(
                ones, mb.astype(jnp.bfloat16), (((1,), (0,)), ((), ())),
                preferred_element_type=jnp.float32)
        parts.append(accf)
    acc = jnp.concatenate(parts, axis=1).astype(jnp.int32)

    r = jnp.reshape(acc, (NCHUNK, CH))
    woff = lax.broadcasted_iota(jnp.int32, (NW, 1, 1), 0) * N
    idx_ref[...] = r[None, :, :] + woff


def _rank_indices(x_flat):
    return pl.pallas_call(
        _rank_body,
        out_shape=jax.ShapeDtypeStruct((NW, NCHUNK, CH), jnp.int32),
    )(x_flat.reshape(1, N))


def _scatter_body(yflat, idxmat, out, idx_v, buf0, buf1, buf2, buf3,
                  gsem0, gsem1, gsem2, gsem3, osem0, osem1, osem2, osem3):
    w = lax.axis_index("s") * NC + lax.axis_index("c")
    pltpu.sync_copy(idxmat.at[w], idx_v)

    bufs = (buf0, buf1, buf2, buf3)
    gsems = (gsem0, gsem1, gsem2, gsem3)
    osems = (osem0, osem1, osem2, osem3)

    def fire_g(c, j):
        pltpu.async_copy(yflat.at[pl.ds(w * N + c * CH, CH)], bufs[j],
                         gsems[j])

    def wait_g(c, j):
        pltpu.make_async_copy(yflat.at[pl.ds(w * N + c * CH, CH)], bufs[j],
                              gsems[j]).wait()

    def fire_o(c, j):
        pltpu.async_copy(bufs[j], out.at[idx_v.at[c]], osems[j])

    def wait_o(c, j):
        pltpu.make_async_copy(bufs[j], out.at[idx_v.at[c]], osems[j]).wait()

    for j in range(NBUF):
        fire_g(j, j)

    def body(i, _):
        c = i * NBUF
        for j in range(NBUF):
            wait_g(c + j, j)
            fire_o(c + j, j)
        for j in range(NBUF):
            wait_o(c + j, j)

            @pl.when(i < NITER - 1)
            def _():
                fire_g(c + NBUF + j, j)
        return 0

    lax.fori_loop(0, NITER, body, 0)


def _scatter_rows(yflat, idxmat):
    mesh = plsc.VectorSubcoreMesh(core_axis_name="c", subcore_axis_name="s")
    return pl.kernel(
        _scatter_body,
        out_type=jax.ShapeDtypeStruct((NW * N, D), jnp.float32),
        mesh=mesh,
        scratch_types=(
            [pltpu.VMEM((NCHUNK, CH), jnp.int32)]
            + [pltpu.VMEM((CH, D), jnp.float32)] * NBUF
            + [pltpu.SemaphoreType.DMA] * (2 * NBUF)
        ),
    )(yflat, idxmat)


def kernel(x, y):
    idxmat = _rank_indices(x.reshape(N))
    yflat = y.reshape(NW * N, D)
    out = _scatter_rows(yflat, idxmat.reshape(NW, NCHUNK, CH))
    return out.reshape(y.shape)


# speedup vs baseline: 1.0288x; 1.0068x over previous
import jax
import jax.numpy as jnp
from jax import lax
from jax.experimental import pallas as pl
from jax.experimental.pallas import tpu as pltpu
from jax.experimental.pallas import tpu_sc as plsc

N = 4096
D = 128
NC, NS = 2, 16
NW = NC * NS
BLK = 512
CH = 128
NCHUNK = N // CH
NBUF = 4
NITER = NCHUNK // NBUF


def _rank_body(xrow_ref, idx_ref):
    xrow = xrow_ref[...]
    xcol = jnp.transpose(xrow)

    ones = jnp.ones((1, BLK), jnp.bfloat16)
    gi_l = lax.broadcasted_iota(jnp.int32, (BLK, BLK), 0)
    gj_l = lax.broadcasted_iota(jnp.int32, (BLK, BLK), 1)
    tri = gi_l < gj_l
    NB = N // BLK
    parts = []
    for bj in range(NB):
        xr = xrow[:, bj * BLK:(bj + 1) * BLK]
        accf = jnp.zeros((1, BLK), jnp.float32)
        for bi in range(NB):
            a = xcol[bi * BLK:(bi + 1) * BLK, :]
            if bi < bj:
                mb = a <= xr
            elif bi > bj:
                mb = a < xr
            else:
                mb = (a < xr) | ((a == xr) & tri)
            accf = accf + jax.lax.dot_general(
                ones, mb.astype(jnp.bfloat16), (((1,), (0,)), ((), ())),
                preferred_element_type=jnp.float32)
        parts.append(accf)
    acc = jnp.concatenate(parts, axis=1).astype(jnp.int32)

    r = jnp.reshape(acc, (NCHUNK, CH))
    woff = lax.broadcasted_iota(jnp.int32, (NW, 1, 1), 0) * N
    idx_ref[...] = r[None, :, :] + woff


def _rank_indices(x_flat):
    return pl.pallas_call(
        _rank_body,
        out_shape=jax.ShapeDtypeStruct((NW, NCHUNK, CH), jnp.int32),
    )(x_flat.reshape(1, N))


def _scatter_body(yflat, idxmat, out, idx_v, buf0, buf1, buf2, buf3,
                  gsem0, gsem1, gsem2, gsem3, osem0, osem1, osem2, osem3):
    w = lax.axis_index("s") * NC + lax.axis_index("c")
    pltpu.sync_copy(idxmat.at[w], idx_v)

    bufs = (buf0, buf1, buf2, buf3)
    gsems = (gsem0, gsem1, gsem2, gsem3)
    osems = (osem0, osem1, osem2, osem3)

    def fire_g(c, j):
        pltpu.async_copy(yflat.at[pl.ds(w * N + c * CH, CH)], bufs[j],
                         gsems[j])

    def wait_g(c, j):
        pltpu.make_async_copy(yflat.at[pl.ds(w * N + c * CH, CH)], bufs[j],
                              gsems[j]).wait()

    def fire_o(c, j):
        pltpu.async_copy(bufs[j], out.at[idx_v.at[c]], osems[j])

    def wait_o(c, j):
        pltpu.make_async_copy(bufs[j], out.at[idx_v.at[c]], osems[j]).wait()

    for j in range(NBUF):
        fire_g(j, j)

    def body(i, _):
        c = i * NBUF
        for j in range(NBUF):
            wait_g(c + j, j)
            fire_o(c + j, j)
        for j in range(NBUF):
            wait_o(c + j, j)

            @pl.when(i < NITER - 1)
            def _():
                fire_g(c + NBUF + j, j)
        return 0

    lax.fori_loop(0, NITER, body, 0)


def _scatter_rows(yflat, idxmat):
    mesh = plsc.VectorSubcoreMesh(core_axis_name="c", subcore_axis_name="s")
    return pl.kernel(
        _scatter_body,
        out_type=jax.ShapeDtypeStruct((NW * N, D), jnp.float32),
        mesh=mesh,
        scratch_types=(
            [pltpu.VMEM((NCHUNK, CH), jnp.int32)]
            + [pltpu.VMEM((CH, D), jnp.float32)] * NBUF
            + [pltpu.SemaphoreType.DMA] * (2 * NBUF)
        ),
    )(yflat, idxmat)


def kernel(x, y):
    idxmat = _rank_indices(x.reshape(N))
    yflat = y.reshape(NW * N, D)
    out = _scatter_rows(yflat, idxmat)
    return out.reshape(y.shape)
